# baseline (speedup 1.0000x reference)
import functools

import jax
import jax.numpy as jnp
from jax import lax
from jax.experimental import pallas as pl
from jax.experimental.pallas import tpu as pltpu
from jax.experimental.pallas import tpu_sc as plsc

N_SO = 1024
N_E = 64
RANK = 16
BATCH = 4096
TOTAL = BATCH * N_E

PACK = 128 // (2 * RANK)
NW = 32
CHUNK = TOTAL // NW
W = 1024
WC = W // PACK
NSTEP = CHUNK // W
BB = 256


def _sc_gather(occ2, AP):
    mesh = plsc.VectorSubcoreMesh(core_axis_name="core", subcore_axis_name="subcore")

    @functools.partial(
        pl.kernel,
        out_type=jax.ShapeDtypeStruct((TOTAL // PACK, 128), jnp.float32),
        mesh=mesh,
        compiler_params=pltpu.CompilerParams(use_tc_tiling_on_sc=False),
        scratch_types=(
            [pltpu.VMEM((WC,), jnp.int32) for _ in range(2 * PACK)]
            + [pltpu.VMEM((WC, 2 * RANK), jnp.float32) for _ in range(2 * PACK)]
            + [pltpu.SemaphoreType.DMA for _ in range(6)]
        ),
    )
    def k(occ_hbm, ap_hbm, o_hbm, *scratch):
        idx_sets = [scratch[0:PACK], scratch[PACK:2 * PACK]]
        g_sets = [scratch[2 * PACK:3 * PACK], scratch[3 * PACK:4 * PACK]]
        isem = scratch[4 * PACK:4 * PACK + 2]
        gsem = scratch[4 * PACK + 2:4 * PACK + 4]
        osem = scratch[4 * PACK + 4:4 * PACK + 6]
        wid = lax.axis_index("subcore") * 2 + lax.axis_index("core")
        base = wid * CHUNK

        def issue_idx(s):
            st = s % 2
            off = base + s * W
            return [pltpu.async_copy(occ_hbm.at[pl.ds(off + c * WC, WC)],
                                     idx_sets[st][c], isem[st])
                    for c in range(PACK)]

        def issue_gather(s):
            st = s % 2
            return [pltpu.async_copy(ap_hbm.at[idx_sets[st][c]],
                                     g_sets[st][c], gsem[st])
                    for c in range(PACK)]

        def issue_out(s):
            st = s % 2
            off = base + s * W
            return [pltpu.async_copy(
                        g_sets[st][c],
                        o_hbm.at[pl.ds(off // PACK, W // PACK),
                                 pl.ds(2 * RANK * c, 2 * RANK)],
                        osem[st])
                    for c in range(PACK)]

        icps = {0: issue_idx(0)}
        for cp in icps[0]:
            cp.wait()
        gcps = {0: issue_gather(0)}
        icps[1] = issue_idx(1)
        ocps = {}
        for s in range(NSTEP):
            for cp in gcps[s]:
                cp.wait()
            ocps[s] = issue_out(s)
            if s + 2 < NSTEP:
                icps[s + 2] = issue_idx(s + 2)
            if s + 1 < NSTEP:
                for cp in icps[s + 1]:
                    cp.wait()
                if s >= 1:
                    for cp in ocps[s - 1]:
                        cp.wait()
                gcps[s + 1] = issue_gather(s + 1)
        for cp in ocps[NSTEP - 2] + ocps[NSTEP - 1]:
            cp.wait()

    return k(occ2, AP)


def _tc_contract(gap, bdiag, ident):
    rows = BB * N_E // PACK
    grp = N_E // PACK

    def body(gap_ref, bd_ref, id_ref, out_ref):
        g = gap_ref[...]
        colsum = jnp.sum(g.reshape(BB, grp, 128), axis=1)
        w = colsum[:, RANK:2 * RANK]
        for c in range(1, PACK):
            w = w + colsum[:, 2 * RANK * c + RANK:2 * RANK * (c + 1)]
        w = w * (1.0 / N_E)
        wt = jnp.concatenate([w] * (128 // RANK), axis=1)
        wrep = jnp.broadcast_to(wt[:, None, :], (BB, grp, 128)).reshape(rows, 128)
        s = g * wrep
        o = jnp.dot(s, bd_ref[...], preferred_element_type=jnp.float32)
        o3 = o.reshape(BB, grp, PACK * N_E)
        ident_b = id_ref[...]
        for j in range(grp):
            ot = jax.lax.dot_general(
                o3[:, j, :], ident_b,
                (((0,), (0,)), ((), ())),
                preferred_element_type=jnp.float32)
            for c in range(PACK):
                out_ref[grp * c + j, :, :] = ot[N_E * c:N_E * (c + 1), :]

    return pl.pallas_call(
        body,
        grid=(BATCH // BB,),
        in_specs=[
            pl.BlockSpec((rows, 128), lambda i: (i, 0)),
            pl.BlockSpec((128, PACK * N_E), lambda i: (0, 0)),
            pl.BlockSpec((BB, BB), lambda i: (0, 0)),
        ],
        out_specs=pl.BlockSpec((N_E, N_E, BB), lambda i: (0, 0, i)),
        out_shape=jax.ShapeDtypeStruct((N_E, N_E, BATCH), jnp.float32),
    )(gap, bdiag, ident)


def kernel(occ_so, A, B, P):
    occ1 = (occ_so.astype(jnp.int32)
            .reshape(BATCH, PACK, N_E // PACK)
            .transpose(0, 2, 1)
            .reshape(TOTAL))
    occ2 = (occ1.reshape(TOTAL // W, WC, PACK)
            .swapaxes(1, 2)
            .reshape(TOTAL))
    AP = jnp.concatenate([A, P], axis=1)
    gap = _sc_gather(occ2, AP)
    bt0 = jnp.concatenate([B.T, jnp.zeros((RANK, N_E), jnp.float32)], axis=0)
    bdiag = jnp.kron(jnp.eye(PACK, dtype=jnp.float32), bt0)
    ident = jnp.eye(BB, dtype=jnp.float32)
    out_t = _tc_contract(gap, bdiag, ident)
    return jnp.transpose(out_t, (2, 0, 1))


# speedup vs baseline: 1.0760x; 1.0760x over previous
import functools

import jax
import jax.numpy as jnp
from jax import lax
from jax.experimental import pallas as pl
from jax.experimental.pallas import tpu as pltpu
from jax.experimental.pallas import tpu_sc as plsc

N_SO = 1024
N_E = 64
RANK = 16
BATCH = 4096
TOTAL = BATCH * N_E

PACK = 128 // (2 * RANK)
NW = 32
CHUNK = TOTAL // NW
W = 2048
WC = W // PACK
WB = W // N_E
NSTEP = CHUNK // W
BB = 128


def _sc_gather(occ2, AP):
    mesh = plsc.VectorSubcoreMesh(core_axis_name="core", subcore_axis_name="subcore")

    @functools.partial(
        pl.kernel,
        out_type=jax.ShapeDtypeStruct((TOTAL // PACK, 128), jnp.float32),
        mesh=mesh,
        compiler_params=pltpu.CompilerParams(use_tc_tiling_on_sc=False),
        scratch_types=(
            [pltpu.VMEM((WC,), jnp.int32) for _ in range(PACK)]
            + [pltpu.VMEM((WC, 2 * RANK), jnp.float32) for _ in range(PACK)]
            + [pltpu.SemaphoreType.DMA, pltpu.SemaphoreType.DMA,
               pltpu.SemaphoreType.DMA]
        ),
    )
    def k(occ_hbm, ap_hbm, o_hbm, *scratch):
        idx_refs = scratch[0:PACK]
        g_refs = scratch[PACK:2 * PACK]
        isem, gsem, osem = scratch[2 * PACK:2 * PACK + 3]
        wid = lax.axis_index("subcore") * 2 + lax.axis_index("core")
        base = wid * CHUNK

        @pl.loop(0, NSTEP)
        def _(s):
            off = base + s * W
            icps = [
                pltpu.async_copy(occ_hbm.at[pl.ds(off + c * WC, WC)],
                                 idx_refs[c], isem)
                for c in range(PACK)
            ]
            for cp in icps:
                cp.wait()
            cps = [
                pltpu.async_copy(ap_hbm.at[idx_refs[c]], g_refs[c], gsem)
                for c in range(PACK)
            ]
            for cp in cps:
                cp.wait()
            ocps = [
                pltpu.async_copy(
                    g_refs[c],
                    o_hbm.at[pl.ds(off // PACK, W // PACK),
                             pl.ds(2 * RANK * c, 2 * RANK)],
                    osem)
                for c in range(PACK)
            ]
            for cp in ocps:
                cp.wait()

    return k(occ2, AP)


def _tc_contract(gap, bdiag, ident):
    rows = BB * N_E // PACK
    grp = N_E // PACK

    def body(gap_ref, bd_ref, id_ref, out_ref):
        g = gap_ref[...]
        colsum = jnp.sum(g.reshape(BB, grp, 128), axis=1)
        w = colsum[:, RANK:2 * RANK]
        for c in range(1, PACK):
            w = w + colsum[:, 2 * RANK * c + RANK:2 * RANK * (c + 1)]
        w = w * (1.0 / N_E)
        wt = jnp.concatenate([w] * (128 // RANK), axis=1)
        wrep = jnp.broadcast_to(wt[:, None, :], (BB, grp, 128)).reshape(rows, 128)
        s = g * wrep
        o = jnp.dot(s, bd_ref[...], preferred_element_type=jnp.float32)
        o3 = o.reshape(BB, grp, PACK * N_E)
        ident_b = id_ref[...]
        for j in range(grp):
            ot = jax.lax.dot_general(
                o3[:, j, :], ident_b,
                (((0,), (0,)), ((), ())),
                preferred_element_type=jnp.float32)
            for c in range(PACK):
                out_ref[grp * c + j, :, :] = ot[N_E * c:N_E * (c + 1), :]

    return pl.pallas_call(
        body,
        grid=(BATCH // BB,),
        in_specs=[
            pl.BlockSpec((rows, 128), lambda i: (i, 0)),
            pl.BlockSpec((128, PACK * N_E), lambda i: (0, 0)),
            pl.BlockSpec((BB, BB), lambda i: (0, 0)),
        ],
        out_specs=pl.BlockSpec((N_E, N_E, BB), lambda i: (0, 0, i)),
        out_shape=jax.ShapeDtypeStruct((N_E, N_E, BATCH), jnp.float32),
    )(gap, bdiag, ident)


def kernel(occ_so, A, B, P):
    occ1 = (occ_so.astype(jnp.int32)
            .reshape(BATCH, PACK, N_E // PACK)
            .transpose(0, 2, 1)
            .reshape(TOTAL))
    occ2 = (occ1.reshape(TOTAL // W, WC, PACK)
            .swapaxes(1, 2)
            .reshape(TOTAL))
    AP = jnp.concatenate([A, P], axis=1)
    gap = _sc_gather(occ2, AP)
    bt0 = jnp.concatenate([B.T, jnp.zeros((RANK, N_E), jnp.float32)], axis=0)
    bdiag = jnp.kron(jnp.eye(PACK, dtype=jnp.float32), bt0)
    ident = jnp.eye(BB, dtype=jnp.float32)
    out_t = _tc_contract(gap, bdiag, ident)
    return jnp.transpose(out_t, (2, 0, 1))


# speedup vs baseline: 1.1574x; 1.0757x over previous
import functools

import jax
import jax.numpy as jnp
from jax import lax
from jax.experimental import pallas as pl
from jax.experimental.pallas import tpu as pltpu
from jax.experimental.pallas import tpu_sc as plsc

N_SO = 1024
N_E = 64
RANK = 16
BATCH = 4096
TOTAL = BATCH * N_E

PACK = 128 // (2 * RANK)
NW = 32
CHUNK = TOTAL // NW
W = 2048
WC = W // PACK
WB = W // N_E
NSTEP = CHUNK // W
BB = 128
NHALF = 2
HALF = TOTAL // NHALF


def _sc_gather(occ2, AP):
    mesh = plsc.VectorSubcoreMesh(core_axis_name="core", subcore_axis_name="subcore")

    chunk = occ2.shape[0] // NW
    nstep = chunk // W

    @functools.partial(
        pl.kernel,
        out_type=jax.ShapeDtypeStruct((occ2.shape[0] // PACK, 128), jnp.float32),
        mesh=mesh,
        compiler_params=pltpu.CompilerParams(use_tc_tiling_on_sc=False),
        scratch_types=(
            [pltpu.VMEM((WC,), jnp.int32) for _ in range(PACK)]
            + [pltpu.VMEM((WC, 2 * RANK), jnp.float32) for _ in range(PACK)]
            + [pltpu.SemaphoreType.DMA, pltpu.SemaphoreType.DMA,
               pltpu.SemaphoreType.DMA]
        ),
    )
    def k(occ_hbm, ap_hbm, o_hbm, *scratch):
        idx_refs = scratch[0:PACK]
        g_refs = scratch[PACK:2 * PACK]
        isem, gsem, osem = scratch[2 * PACK:2 * PACK + 3]
        wid = lax.axis_index("subcore") * 2 + lax.axis_index("core")
        base = wid * chunk

        @pl.loop(0, nstep)
        def _(s):
            off = base + s * W
            icps = [
                pltpu.async_copy(occ_hbm.at[pl.ds(off + c * WC, WC)],
                                 idx_refs[c], isem)
                for c in range(PACK)
            ]
            for cp in icps:
                cp.wait()
            cps = [
                pltpu.async_copy(ap_hbm.at[idx_refs[c]], g_refs[c], gsem)
                for c in range(PACK)
            ]
            for cp in cps:
                cp.wait()
            ocps = [
                pltpu.async_copy(
                    g_refs[c],
                    o_hbm.at[pl.ds(off // PACK, W // PACK),
                             pl.ds(2 * RANK * c, 2 * RANK)],
                    osem)
                for c in range(PACK)
            ]
            for cp in ocps:
                cp.wait()

    return k(occ2, AP)


def _tc_contract(gap, bdiag, ident, half, partial):
    rows = BB * N_E // PACK
    grp = N_E // PACK

    nblk = (BATCH // NHALF) // BB

    def body(gap_ref, bd_ref, id_ref, *rest):
        out_ref = rest[-1]
        g = gap_ref[...]
        colsum = jnp.sum(g.reshape(BB, grp, 128), axis=1)
        w = colsum[:, RANK:2 * RANK]
        for c in range(1, PACK):
            w = w + colsum[:, 2 * RANK * c + RANK:2 * RANK * (c + 1)]
        w = w * (1.0 / N_E)
        wt = jnp.concatenate([w] * (128 // RANK), axis=1)
        wrep = jnp.broadcast_to(wt[:, None, :], (BB, grp, 128)).reshape(rows, 128)
        s = g * wrep
        o = jnp.dot(s, bd_ref[...], preferred_element_type=jnp.float32)
        o3 = o.reshape(BB, grp, PACK * N_E)
        ident_b = id_ref[...]
        for j in range(grp):
            ot = jax.lax.dot_general(
                o3[:, j, :], ident_b,
                (((0,), (0,)), ((), ())),
                preferred_element_type=jnp.float32)
            for c in range(PACK):
                out_ref[grp * c + j, :, :] = ot[N_E * c:N_E * (c + 1), :]

    in_specs = [
        pl.BlockSpec((rows, 128), lambda i: (i, 0)),
        pl.BlockSpec((128, PACK * N_E), lambda i: (0, 0)),
        pl.BlockSpec((BB, BB), lambda i: (0, 0)),
    ]
    out_spec = pl.BlockSpec((N_E, N_E, BB),
                            lambda i, h=half: (0, 0, i + h * nblk))
    out_shape = jax.ShapeDtypeStruct((N_E, N_E, BATCH), jnp.float32)
    if partial is None:
        return pl.pallas_call(
            body, grid=(nblk,), in_specs=in_specs,
            out_specs=out_spec, out_shape=out_shape,
        )(gap, bdiag, ident)
    return pl.pallas_call(
        body, grid=(nblk,),
        in_specs=in_specs + [pl.BlockSpec(memory_space=pl.ANY)],
        out_specs=out_spec, out_shape=out_shape,
        input_output_aliases={3: 0},
    )(gap, bdiag, ident, partial)


def kernel(occ_so, A, B, P):
    occ1 = (occ_so.astype(jnp.int32)
            .reshape(BATCH, PACK, N_E // PACK)
            .transpose(0, 2, 1)
            .reshape(TOTAL))
    occ2 = (occ1.reshape(TOTAL // W, WC, PACK)
            .swapaxes(1, 2)
            .reshape(TOTAL))
    AP = jnp.concatenate([A, P], axis=1)
    bt0 = jnp.concatenate([B.T, jnp.zeros((RANK, N_E), jnp.float32)], axis=0)
    bdiag = jnp.kron(jnp.eye(PACK, dtype=jnp.float32), bt0)
    ident = jnp.eye(BB, dtype=jnp.float32)
    out_t = None
    for h in range(NHALF):
        gap_h = _sc_gather(occ2[h * HALF:(h + 1) * HALF], AP)
        out_t = _tc_contract(gap_h, bdiag, ident, h, out_t)
    return jnp.transpose(out_t, (2, 0, 1))


# speedup vs baseline: 1.2354x; 1.0674x over previous
import functools

import jax
import jax.numpy as jnp
from jax import lax
from jax.experimental import pallas as pl
from jax.experimental.pallas import tpu as pltpu
from jax.experimental.pallas import tpu_sc as plsc

N_SO = 1024
N_E = 64
RANK = 16
BATCH = 4096
TOTAL = BATCH * N_E

PACK = 128 // (2 * RANK)
NW = 32
CHUNK = TOTAL // NW
W = 2048
WC = W // PACK
WB = W // N_E
NSTEP = CHUNK // W
BB = 128
NHALF = 4
HALF = TOTAL // NHALF


def _sc_gather(occ2, AP):
    mesh = plsc.VectorSubcoreMesh(core_axis_name="core", subcore_axis_name="subcore")

    chunk = occ2.shape[0] // NW
    nstep = chunk // W

    @functools.partial(
        pl.kernel,
        out_type=jax.ShapeDtypeStruct((occ2.shape[0] // PACK, 128), jnp.float32),
        mesh=mesh,
        compiler_params=pltpu.CompilerParams(use_tc_tiling_on_sc=False),
        scratch_types=(
            [pltpu.VMEM((WC,), jnp.int32) for _ in range(PACK)]
            + [pltpu.VMEM((WC, 2 * RANK), jnp.float32) for _ in range(PACK)]
            + [pltpu.SemaphoreType.DMA, pltpu.SemaphoreType.DMA,
               pltpu.SemaphoreType.DMA]
        ),
    )
    def k(occ_hbm, ap_hbm, o_hbm, *scratch):
        idx_refs = scratch[0:PACK]
        g_refs = scratch[PACK:2 * PACK]
        isem, gsem, osem = scratch[2 * PACK:2 * PACK + 3]
        wid = lax.axis_index("subcore") * 2 + lax.axis_index("core")
        base = wid * chunk

        @pl.loop(0, nstep)
        def _(s):
            off = base + s * W
            icps = [
                pltpu.async_copy(occ_hbm.at[pl.ds(off + c * WC, WC)],
                                 idx_refs[c], isem)
                for c in range(PACK)
            ]
            for cp in icps:
                cp.wait()
            cps = [
                pltpu.async_copy(ap_hbm.at[idx_refs[c]], g_refs[c], gsem)
                for c in range(PACK)
            ]
            for cp in cps:
                cp.wait()
            ocps = [
                pltpu.async_copy(
                    g_refs[c],
                    o_hbm.at[pl.ds(off // PACK, W // PACK),
                             pl.ds(2 * RANK * c, 2 * RANK)],
                    osem)
                for c in range(PACK)
            ]
            for cp in ocps:
                cp.wait()

    return k(occ2, AP)


def _tc_contract(gap, bdiag, ident, half, partial):
    rows = BB * N_E // PACK
    grp = N_E // PACK

    nblk = (BATCH // NHALF) // BB

    def body(gap_ref, bd_ref, id_ref, *rest):
        out_ref = rest[-1]
        g = gap_ref[...]
        colsum = jnp.sum(g.reshape(BB, grp, 128), axis=1)
        w = colsum[:, RANK:2 * RANK]
        for c in range(1, PACK):
            w = w + colsum[:, 2 * RANK * c + RANK:2 * RANK * (c + 1)]
        w = w * (1.0 / N_E)
        wt = jnp.concatenate([w] * (128 // RANK), axis=1)
        wrep = jnp.broadcast_to(wt[:, None, :], (BB, grp, 128)).reshape(rows, 128)
        s = g * wrep
        o = jnp.dot(s, bd_ref[...], preferred_element_type=jnp.float32)
        o3 = o.reshape(BB, grp, PACK * N_E)
        ident_b = id_ref[...]
        for j in range(grp):
            ot = jax.lax.dot_general(
                o3[:, j, :], ident_b,
                (((0,), (0,)), ((), ())),
                preferred_element_type=jnp.float32)
            for c in range(PACK):
                out_ref[grp * c + j, :, :] = ot[N_E * c:N_E * (c + 1), :]

    in_specs = [
        pl.BlockSpec((rows, 128), lambda i: (i, 0)),
        pl.BlockSpec((128, PACK * N_E), lambda i: (0, 0)),
        pl.BlockSpec((BB, BB), lambda i: (0, 0)),
    ]
    out_spec = pl.BlockSpec((N_E, N_E, BB),
                            lambda i, h=half: (0, 0, i + h * nblk))
    out_shape = jax.ShapeDtypeStruct((N_E, N_E, BATCH), jnp.float32)
    if partial is None:
        return pl.pallas_call(
            body, grid=(nblk,), in_specs=in_specs,
            out_specs=out_spec, out_shape=out_shape,
        )(gap, bdiag, ident)
    return pl.pallas_call(
        body, grid=(nblk,),
        in_specs=in_specs + [pl.BlockSpec(memory_space=pl.ANY)],
        out_specs=out_spec, out_shape=out_shape,
        input_output_aliases={3: 0},
    )(gap, bdiag, ident, partial)


def kernel(occ_so, A, B, P):
    occ1 = (occ_so.astype(jnp.int32)
            .reshape(BATCH, PACK, N_E // PACK)
            .transpose(0, 2, 1)
            .reshape(TOTAL))
    occ2 = (occ1.reshape(TOTAL // W, WC, PACK)
            .swapaxes(1, 2)
            .reshape(TOTAL))
    AP = jnp.concatenate([A, P], axis=1)
    bt0 = jnp.concatenate([B.T, jnp.zeros((RANK, N_E), jnp.float32)], axis=0)
    bdiag = jnp.kron(jnp.eye(PACK, dtype=jnp.float32), bt0)
    ident = jnp.eye(BB, dtype=jnp.float32)
    out_t = None
    for h in range(NHALF):
        gap_h = _sc_gather(occ2[h * HALF:(h + 1) * HALF], AP)
        out_t = _tc_contract(gap_h, bdiag, ident, h, out_t)
    return jnp.transpose(out_t, (2, 0, 1))
